# Initial kernel scaffold; baseline (speedup 1.0000x reference)
import jax
import jax.numpy as jnp
from jax.experimental import pallas as pl


def kernel(x, relative_position_encoding):
    raise NotImplementedError("write your pallas kernel here")


# speedup vs baseline: 8.1478x; 8.1478x over previous
import jax
import jax.numpy as jnp
from jax.experimental import pallas as pl
from jax.experimental.pallas import tpu as pltpu

D_MODEL = 512
MAX_REL = 32
SEQ_LEN = 512
NB = 2 * SEQ_LEN


def _build_b_body(t_ref, b_ref):
    lo = SEQ_LEN - 1 - MAX_REL
    b_ref[0:480, :] = jnp.broadcast_to(t_ref[0:1, :], (480, D_MODEL))
    b_ref[480:544, :] = t_ref[1:65, :]
    b_ref[544:NB, :] = jnp.broadcast_to(t_ref[64:65, :], (NB - 544, D_MODEL))
    del lo


def _main_body(b_ref, x_ref, o_ref):
    g = pl.program_id(0)
    base = pl.multiple_of(8 * (SEQ_LEN // 8 - 1 - g), 8)
    v = b_ref[pl.ds(base, SEQ_LEN + 8), :]
    for u in range(8):
        o_ref[u] = v[7 - u : 7 - u + SEQ_LEN, :] + x_ref[u]


def kernel(x, relative_position_encoding):
    t = relative_position_encoding
    b = pl.pallas_call(
        _build_b_body,
        out_shape=jax.ShapeDtypeStruct((NB, D_MODEL), jnp.float32),
    )(t)
    out = pl.pallas_call(
        _main_body,
        grid=(SEQ_LEN // 8,),
        in_specs=[
            pl.BlockSpec((NB, D_MODEL), lambda g: (0, 0)),
            pl.BlockSpec((8, 1, D_MODEL), lambda g: (g, 0, 0)),
        ],
        out_specs=pl.BlockSpec((8, SEQ_LEN, D_MODEL), lambda g: (g, 0, 0)),
        out_shape=jax.ShapeDtypeStruct((SEQ_LEN, SEQ_LEN, D_MODEL), jnp.float32),
    )(b, x)
    return out
